# Initial kernel scaffold; baseline (speedup 1.0000x reference)
import jax
import jax.numpy as jnp
from jax.experimental import pallas as pl
from jax.experimental.pallas import tpu as pltpu

_INPUT = 768
_HIDDEN = 8192
_K = 128
_TB = 128

_INT_MIN = jnp.int32(-2147483648)


def _sae_block(x_ref, wt_ref, w_ref, b1_ref, b2_ref, o_ref):
    xc = x_ref[...] - b2_ref[...]
    pre = (
        jnp.dot(xc, wt_ref[...], preferred_element_type=jnp.float32)
        + b1_ref[...]
    )

    bits = jax.lax.bitcast_convert_type(pre, jnp.int32)
    keys = jnp.where(bits < 0, bits ^ jnp.int32(0x7FFFFFFF), bits)

    def body(i, p):
        bit = jnp.left_shift(jnp.int32(1), jnp.int32(31) - i)
        cand_u = p | bit
        cand_s = cand_u ^ _INT_MIN
        cnt = jnp.sum((keys >= cand_s).astype(jnp.int32), axis=1,
                      keepdims=True)
        return jnp.where(cnt >= _K, cand_u, p)

    p = jax.lax.fori_loop(0, 32, body, jnp.zeros((_TB, 1), jnp.int32))
    thresh = p ^ _INT_MIN

    masked = jnp.where(keys >= thresh, pre, 0.0).astype(jnp.bfloat16)
    out = jnp.dot(masked, w_ref[...], preferred_element_type=jnp.float32)
    o_ref[...] = out + b2_ref[...]


def kernel(x, W, WT, b1, b2):
    tokens = x.shape[0]
    w_bf16 = W.astype(jnp.bfloat16)
    b1r = b1.reshape(1, _HIDDEN)
    b2r = b2.reshape(1, _INPUT)
    grid = (tokens // _TB,)
    return pl.pallas_call(
        _sae_block,
        grid=grid,
        in_specs=[
            pl.BlockSpec((_TB, _INPUT), lambda i: (i, 0)),
            pl.BlockSpec((_INPUT, _HIDDEN), lambda i: (0, 0)),
            pl.BlockSpec((_HIDDEN, _INPUT), lambda i: (0, 0)),
            pl.BlockSpec((1, _HIDDEN), lambda i: (0, 0)),
            pl.BlockSpec((1, _INPUT), lambda i: (0, 0)),
        ],
        out_specs=pl.BlockSpec((_TB, _INPUT), lambda i: (i, 0)),
        out_shape=jax.ShapeDtypeStruct((tokens, _INPUT), jnp.float32),
        compiler_params=pltpu.CompilerParams(
            dimension_semantics=("arbitrary",),
        ),
    )(x, WT, w_bf16, b1r, b2r)


# speedup vs baseline: 16.5141x; 16.5141x over previous
import jax
import jax.numpy as jnp
from jax.experimental import pallas as pl
from jax.experimental.pallas import tpu as pltpu

_INPUT = 768
_HIDDEN = 8192
_K = 128
_TB = 128

_INT_MIN = -2147483648


def _sae_block(x_ref, wt_ref, w_ref, b1_ref, b2_ref, o_ref):
    xc = x_ref[...] - b2_ref[...]
    pre = (
        jnp.dot(xc, wt_ref[...], preferred_element_type=jnp.float32)
        + b1_ref[...]
    )

    bits = jax.lax.bitcast_convert_type(pre, jnp.int32)
    keys = jnp.where(bits < 0, bits ^ 0x7FFFFFFF, bits)

    def body(i, p):
        bit = jnp.left_shift(jnp.int32(1), 31 - i)
        cand_u = p | bit
        cand_s = cand_u ^ _INT_MIN
        cnt = jnp.sum((keys >= cand_s).astype(jnp.int32), axis=1,
                      keepdims=True)
        return jnp.where(cnt >= _K, cand_u, p)

    p = jax.lax.fori_loop(0, 32, body, jnp.zeros((_TB, 1), jnp.int32))
    thresh = p ^ _INT_MIN

    masked = jnp.where(keys >= thresh, pre, 0.0).astype(jnp.bfloat16)
    out = jnp.dot(masked, w_ref[...], preferred_element_type=jnp.float32)
    o_ref[...] = out + b2_ref[...]


def kernel(x, W, WT, b1, b2):
    tokens = x.shape[0]
    w_bf16 = W.astype(jnp.bfloat16)
    b1r = b1.reshape(1, _HIDDEN)
    b2r = b2.reshape(1, _INPUT)
    grid = (tokens // _TB,)
    return pl.pallas_call(
        _sae_block,
        grid=grid,
        in_specs=[
            pl.BlockSpec((_TB, _INPUT), lambda i: (i, 0)),
            pl.BlockSpec((_INPUT, _HIDDEN), lambda i: (0, 0)),
            pl.BlockSpec((_HIDDEN, _INPUT), lambda i: (0, 0)),
            pl.BlockSpec((1, _HIDDEN), lambda i: (0, 0)),
            pl.BlockSpec((1, _INPUT), lambda i: (0, 0)),
        ],
        out_specs=pl.BlockSpec((_TB, _INPUT), lambda i: (i, 0)),
        out_shape=jax.ShapeDtypeStruct((tokens, _INPUT), jnp.float32),
        compiler_params=pltpu.CompilerParams(
            dimension_semantics=("arbitrary",),
        ),
    )(x, WT, w_bf16, b1r, b2r)
